# baseline (speedup 1.0000x reference)
import functools

import jax
import jax.numpy as jnp
from jax import lax
from jax.experimental import pallas as pl
from jax.experimental.pallas import tpu as pltpu

LAMBDA_PARAM = 25.0
MU_PARAM = 25.0
NU_PARAM = 1.0
ALPHA = 0.75
EPS = 1e-4
K = 20
L = 576
C = 768
D = 2048
BIG = 3.0e9

_NT = (((1,), (1,)), ((), ()))
_TN = (((0,), (0,)), ((), ()))


def _fiota(shape, dim):
    return lax.broadcasted_iota(jnp.int32, shape, dim).astype(jnp.float32)


def _dot(a, b, dims):
    return lax.dot_general(a, b, dims, precision=lax.Precision.HIGHEST)


def _cdist_sq(za, zb):
    a2 = jnp.sum(za * za, axis=1, keepdims=True)
    ones = jnp.ones((1, za.shape[1]), jnp.float32)
    b2t = _dot(ones, zb * zb, _NT)
    d2 = a2 + b2t - 2.0 * lax.dot_general(za, zb, _NT,
                                          precision=lax.Precision.DEFAULT)
    return jnp.maximum(d2, 1e-12)


def _topk_onehots(nnv4):
    col = _fiota((4, L), 1)
    vals = nnv4
    hits = []
    for _ in range(K):
        m = jnp.min(vals, axis=1, keepdims=True)
        idx = jnp.min(jnp.where(vals == m, col, BIG), axis=1, keepdims=True)
        hit = col == idx
        hits.append(hit.astype(jnp.float32))
        vals = jnp.where(hit, BIG, vals)
    return hits


def _select_pairs(s, dist, nnv, zin, zcand, ddim):
    dsel = _dot(s, dist, (((1,), (ddim,)), ((), ())))
    nnv_sel = _dot(s, nnv, _NT)
    kcol = _fiota((K, L), 1)
    cand_f = jnp.min(jnp.where(dsel == nnv_sel, kcol, BIG), axis=1, keepdims=True)
    t = (kcol == cand_f).astype(jnp.float32)
    xin = _dot(s, zin, (((1,), (0,)), ((), ())))
    xcand = _dot(t, zcand, (((1,), (0,)), ((), ())))
    return xin, xcand


def _match_kernel(za_ref, zb_ref, ga_ref, gb_ref, fa_ref, na_ref):
    za = za_ref[0]
    zb = zb_ref[0]
    ga = ga_ref[0]
    gb = gb_ref[0]

    dist_f = _cdist_sq(za, zb)
    dist_g = _cdist_sq(ga, gb)

    nnv4 = jnp.concatenate([
        lax.transpose(jnp.min(dist_f, axis=1, keepdims=True), (1, 0)),
        jnp.min(dist_f, axis=0, keepdims=True),
        lax.transpose(jnp.min(dist_g, axis=1, keepdims=True), (1, 0)),
        jnp.min(dist_g, axis=0, keepdims=True),
    ], axis=0)
    hits = _topk_onehots(nnv4)

    for p, (dist, ddim, zin, zcand) in enumerate((
            (dist_f, 0, za, zb), (dist_f, 1, zb, za),
            (dist_g, 0, za, zb), (dist_g, 1, zb, za))):
        s = jnp.concatenate([h[p:p + 1] for h in hits], axis=0)
        nnv = nnv4[p:p + 1]
        xin, xcand = _select_pairs(s, dist, nnv, zin, zcand, ddim)
        fa_ref[p, 0] = xin
        na_ref[p, 0] = xcand


def _var_loss(xc, n):
    var = jnp.sum(xc * xc, axis=0, keepdims=True) / (n - 1.0)
    std = jnp.sqrt(var + EPS)
    return jnp.mean(jnp.maximum(1.0 - std, 0.0))


def _cov_loss_direct(xc, n, d):
    m = _dot(xc, xc, _TN)
    msq = m * m
    diag = lax.broadcasted_iota(jnp.int32, msq.shape, 0) == \
        lax.broadcasted_iota(jnp.int32, msq.shape, 1)
    off = jnp.sum(jnp.where(diag, 0.0, msq))
    return off / ((n - 1.0) ** 2 * d)


def _cov_loss_gram(xc, n, d):
    g = _dot(xc, xc, _NT)
    s = jnp.sum(xc * xc, axis=0, keepdims=True)
    off = jnp.sum(g * g) - jnp.sum(s * s)
    return off / ((n - 1.0) ** 2 * d)


def _vicreg_pair(xa, xb, n, d, gram):
    inv = jnp.sum((xa - xb) ** 2) / (n * d)
    xca = xa - jnp.mean(xa, axis=0, keepdims=True)
    xcb = xb - jnp.mean(xb, axis=0, keepdims=True)
    var = 0.5 * (_var_loss(xca, n) + _var_loss(xcb, n))
    covf = _cov_loss_gram if gram else _cov_loss_direct
    cov = covf(xca, n, d) + covf(xcb, n, d)
    return LAMBDA_PARAM * inv + MU_PARAM * var + NU_PARAM * cov


def _vicreg_kernel(fa_ref, na_ref, g0_ref, g1_ref, out_ref, acc_ref):
    p = pl.program_id(0)

    @pl.when(p == 0)
    def _():
        acc_ref[0] = ALPHA * _vicreg_pair(g0_ref[...], g1_ref[...], 64.0, float(D), True)

    n = float(64 * K)
    acc_ref[0] += (1.0 - ALPHA) * 0.5 * _vicreg_pair(
        fa_ref[0], na_ref[0], n, float(C), False)

    @pl.when(p == 3)
    def _():
        out_ref[0] = acc_ref[0]


@jax.jit
def _vicregl(z_global0, z_global1, z_local0, z_local1, grid0, grid1):
    B = z_local0.shape[0]
    za = z_local0.reshape(B, L, C)
    zb = z_local1.reshape(B, L, C)
    ga = grid0.reshape(B, L, 2)
    gb = grid1.reshape(B, L, 2)

    fa, na = pl.pallas_call(
        _match_kernel,
        grid=(B,),
        in_specs=[
            pl.BlockSpec((1, L, C), lambda b: (b, 0, 0)),
            pl.BlockSpec((1, L, C), lambda b: (b, 0, 0)),
            pl.BlockSpec((1, L, 2), lambda b: (b, 0, 0)),
            pl.BlockSpec((1, L, 2), lambda b: (b, 0, 0)),
        ],
        out_specs=[
            pl.BlockSpec((4, 1, K, C), lambda b: (0, b, 0, 0)),
            pl.BlockSpec((4, 1, K, C), lambda b: (0, b, 0, 0)),
        ],
        out_shape=[
            jax.ShapeDtypeStruct((4, B, K, C), jnp.float32),
            jax.ShapeDtypeStruct((4, B, K, C), jnp.float32),
        ],
    )(za, zb, ga, gb)

    fa = fa.reshape(4, B * K, C)
    na = na.reshape(4, B * K, C)

    out = pl.pallas_call(
        _vicreg_kernel,
        grid=(4,),
        in_specs=[
            pl.BlockSpec((1, B * K, C), lambda p: (p, 0, 0)),
            pl.BlockSpec((1, B * K, C), lambda p: (p, 0, 0)),
            pl.BlockSpec((B, D), lambda p: (0, 0)),
            pl.BlockSpec((B, D), lambda p: (0, 0)),
        ],
        out_specs=pl.BlockSpec(memory_space=pltpu.SMEM),
        out_shape=jax.ShapeDtypeStruct((1,), jnp.float32),
        scratch_shapes=[pltpu.SMEM((1,), jnp.float32)],
    )(fa, na, z_global0, z_global1)

    return out[0]


def kernel(z_global0, z_global1, z_local0, z_local1, grid0, grid1):
    return _vicregl(z_global0, z_global1, z_local0, z_local1, grid0, grid1)


# speedup vs baseline: 1.0041x; 1.0041x over previous
import functools

import jax
import jax.numpy as jnp
from jax import lax
from jax.experimental import pallas as pl
from jax.experimental.pallas import tpu as pltpu

LAMBDA_PARAM = 25.0
MU_PARAM = 25.0
NU_PARAM = 1.0
ALPHA = 0.75
EPS = 1e-4
K = 20
L = 576
C = 768
D = 2048
BIG = 3.0e9

_NT = (((1,), (1,)), ((), ()))
_TN = (((0,), (0,)), ((), ()))


def _fiota(shape, dim):
    return lax.broadcasted_iota(jnp.int32, shape, dim).astype(jnp.float32)


def _dot(a, b, dims):
    return lax.dot_general(a, b, dims, precision=lax.Precision.HIGHEST)


def _cdist_sq(za, zb):
    a2 = jnp.sum(za * za, axis=1, keepdims=True)
    ones = jnp.ones((1, za.shape[1]), jnp.float32)
    b2t = _dot(ones, zb * zb, _NT)
    d2 = a2 + b2t - 2.0 * lax.dot_general(za, zb, _NT,
                                          precision=lax.Precision.DEFAULT)
    return jnp.maximum(d2, 1e-12)


def _topk_onehots(nnv4):
    col = _fiota((4, L), 1)
    vals = nnv4
    hits = []
    for _ in range(K):
        m = jnp.min(vals, axis=1, keepdims=True)
        idx = jnp.min(jnp.where(vals == m, col, BIG), axis=1, keepdims=True)
        hit = col == idx
        hits.append(hit.astype(jnp.float32))
        vals = jnp.where(hit, BIG, vals)
    return hits


def _select_pairs(s, dist, nnv, zin, zcand, ddim):
    dsel = _dot(s, dist, (((1,), (ddim,)), ((), ())))
    nnv_sel = _dot(s, nnv, _NT)
    kcol = _fiota((K, L), 1)
    cand_f = jnp.min(jnp.where(dsel == nnv_sel, kcol, BIG), axis=1, keepdims=True)
    t = (kcol == cand_f).astype(jnp.float32)
    xin = _dot(s, zin, (((1,), (0,)), ((), ())))
    xcand = _dot(t, zcand, (((1,), (0,)), ((), ())))
    return xin, xcand


IPS = 2


def _match_kernel(za_ref, zb_ref, ga_ref, gb_ref, fa_ref, na_ref):
    for i in range(IPS):
        _match_one(za_ref[i], zb_ref[i], ga_ref[i], gb_ref[i],
                   fa_ref, na_ref, i)


def _match_one(za, zb, ga, gb, fa_ref, na_ref, i):
    dist_f = _cdist_sq(za, zb)
    dist_g = _cdist_sq(ga, gb)

    nnv4 = jnp.concatenate([
        lax.transpose(jnp.min(dist_f, axis=1, keepdims=True), (1, 0)),
        jnp.min(dist_f, axis=0, keepdims=True),
        lax.transpose(jnp.min(dist_g, axis=1, keepdims=True), (1, 0)),
        jnp.min(dist_g, axis=0, keepdims=True),
    ], axis=0)
    hits = _topk_onehots(nnv4)

    for p, (dist, ddim, zin, zcand) in enumerate((
            (dist_f, 0, za, zb), (dist_f, 1, zb, za),
            (dist_g, 0, za, zb), (dist_g, 1, zb, za))):
        s = jnp.concatenate([h[p:p + 1] for h in hits], axis=0)
        nnv = nnv4[p:p + 1]
        xin, xcand = _select_pairs(s, dist, nnv, zin, zcand, ddim)
        fa_ref[p, i] = xin
        na_ref[p, i] = xcand


def _var_loss(xc, n):
    var = jnp.sum(xc * xc, axis=0, keepdims=True) / (n - 1.0)
    std = jnp.sqrt(var + EPS)
    return jnp.mean(jnp.maximum(1.0 - std, 0.0))


def _cov_loss_direct(xc, n, d):
    m = _dot(xc, xc, _TN)
    msq = m * m
    diag = lax.broadcasted_iota(jnp.int32, msq.shape, 0) == \
        lax.broadcasted_iota(jnp.int32, msq.shape, 1)
    off = jnp.sum(jnp.where(diag, 0.0, msq))
    return off / ((n - 1.0) ** 2 * d)


def _cov_loss_gram(xc, n, d):
    g = _dot(xc, xc, _NT)
    s = jnp.sum(xc * xc, axis=0, keepdims=True)
    off = jnp.sum(g * g) - jnp.sum(s * s)
    return off / ((n - 1.0) ** 2 * d)


def _vicreg_pair(xa, xb, n, d, gram):
    inv = jnp.sum((xa - xb) ** 2) / (n * d)
    xca = xa - jnp.mean(xa, axis=0, keepdims=True)
    xcb = xb - jnp.mean(xb, axis=0, keepdims=True)
    var = 0.5 * (_var_loss(xca, n) + _var_loss(xcb, n))
    covf = _cov_loss_gram if gram else _cov_loss_direct
    cov = covf(xca, n, d) + covf(xcb, n, d)
    return LAMBDA_PARAM * inv + MU_PARAM * var + NU_PARAM * cov


def _vicreg_kernel(fa_ref, na_ref, g0_ref, g1_ref, out_ref, acc_ref):
    p = pl.program_id(0)

    @pl.when(p == 0)
    def _():
        acc_ref[0] = ALPHA * _vicreg_pair(g0_ref[...], g1_ref[...], 64.0, float(D), True)

    n = float(64 * K)
    acc_ref[0] += (1.0 - ALPHA) * 0.5 * _vicreg_pair(
        fa_ref[0], na_ref[0], n, float(C), False)

    @pl.when(p == 3)
    def _():
        out_ref[0] = acc_ref[0]


@jax.jit
def _vicregl(z_global0, z_global1, z_local0, z_local1, grid0, grid1):
    B = z_local0.shape[0]
    za = z_local0.reshape(B, L, C)
    zb = z_local1.reshape(B, L, C)
    ga = grid0.reshape(B, L, 2)
    gb = grid1.reshape(B, L, 2)

    fa, na = pl.pallas_call(
        _match_kernel,
        grid=(B // IPS,),
        in_specs=[
            pl.BlockSpec((IPS, L, C), lambda b: (b, 0, 0)),
            pl.BlockSpec((IPS, L, C), lambda b: (b, 0, 0)),
            pl.BlockSpec((IPS, L, 2), lambda b: (b, 0, 0)),
            pl.BlockSpec((IPS, L, 2), lambda b: (b, 0, 0)),
        ],
        out_specs=[
            pl.BlockSpec((4, IPS, K, C), lambda b: (0, b, 0, 0)),
            pl.BlockSpec((4, IPS, K, C), lambda b: (0, b, 0, 0)),
        ],
        out_shape=[
            jax.ShapeDtypeStruct((4, B, K, C), jnp.float32),
            jax.ShapeDtypeStruct((4, B, K, C), jnp.float32),
        ],
    )(za, zb, ga, gb)

    fa = fa.reshape(4, B * K, C)
    na = na.reshape(4, B * K, C)

    out = pl.pallas_call(
        _vicreg_kernel,
        grid=(4,),
        in_specs=[
            pl.BlockSpec((1, B * K, C), lambda p: (p, 0, 0)),
            pl.BlockSpec((1, B * K, C), lambda p: (p, 0, 0)),
            pl.BlockSpec((B, D), lambda p: (0, 0)),
            pl.BlockSpec((B, D), lambda p: (0, 0)),
        ],
        out_specs=pl.BlockSpec(memory_space=pltpu.SMEM),
        out_shape=jax.ShapeDtypeStruct((1,), jnp.float32),
        scratch_shapes=[pltpu.SMEM((1,), jnp.float32)],
    )(fa, na, z_global0, z_global1)

    return out[0]


def kernel(z_global0, z_global1, z_local0, z_local1, grid0, grid1):
    return _vicregl(z_global0, z_global1, z_local0, z_local1, grid0, grid1)


# speedup vs baseline: 1.5714x; 1.5649x over previous
import functools

import jax
import jax.numpy as jnp
from jax import lax
from jax.experimental import pallas as pl
from jax.experimental.pallas import tpu as pltpu

LAMBDA_PARAM = 25.0
MU_PARAM = 25.0
NU_PARAM = 1.0
ALPHA = 0.75
EPS = 1e-4
K = 20
L = 576
C = 768
D = 2048
BIG = 3.0e9

_NT = (((1,), (1,)), ((), ()))
_TN = (((0,), (0,)), ((), ()))


def _fiota(shape, dim):
    return lax.broadcasted_iota(jnp.int32, shape, dim).astype(jnp.float32)


def _dot(a, b, dims):
    return lax.dot_general(a, b, dims, precision=lax.Precision.HIGHEST)


def _ddot(a, b, dims):
    return lax.dot_general(a, b, dims, precision=lax.Precision.DEFAULT)


def _split16(x):
    hi = x.astype(jnp.bfloat16).astype(jnp.float32)
    return hi, x - hi


def _sdot(a, bh, bl, dims):
    return _ddot(a, bh, dims) + _ddot(a, bl, dims)


def _cdist_sq(za, zb):
    a2 = jnp.sum(za * za, axis=1, keepdims=True)
    ones = jnp.ones((1, za.shape[1]), jnp.float32)
    b2t = _dot(ones, zb * zb, _NT)
    d2 = a2 + b2t - 2.0 * lax.dot_general(za, zb, _NT,
                                          precision=lax.Precision.DEFAULT)
    return jnp.maximum(d2, 1e-12)


def _topk_onehots(nnv4):
    col = _fiota((4, L), 1)
    vals = nnv4
    hits = []
    for _ in range(K):
        m = jnp.min(vals, axis=1, keepdims=True)
        idx = jnp.min(jnp.where(vals == m, col, BIG), axis=1, keepdims=True)
        hit = col == idx
        hits.append(hit.astype(jnp.float32))
        vals = jnp.where(hit, BIG, vals)
    return hits


def _select_pairs(s, disth, distl, zin, zcand, ddim):
    dims = (((1,), (ddim,)), ((), ()))
    dsel = _sdot(s, disth, distl, dims)
    nnv_sel = jnp.min(dsel, axis=1, keepdims=True)
    kcol = _fiota((K, L), 1)
    cand_f = jnp.min(jnp.where(dsel == nnv_sel, kcol, BIG), axis=1, keepdims=True)
    t = (kcol == cand_f).astype(jnp.float32)
    gdims = (((1,), (0,)), ((), ()))
    xin = _sdot(s, zin[0], zin[1], gdims)
    xcand = _sdot(t, zcand[0], zcand[1], gdims)
    return xin, xcand


IPS = 2


def _match_kernel(za_ref, zb_ref, ga_ref, gb_ref, fa_ref, na_ref):
    for i in range(IPS):
        _match_one(za_ref[i], zb_ref[i], ga_ref[i], gb_ref[i],
                   fa_ref, na_ref, i)


def _match_one(za, zb, ga, gb, fa_ref, na_ref, i):
    dist_f = _cdist_sq(za, zb)
    dist_g = _cdist_sq(ga, gb)

    nnv4 = jnp.concatenate([
        lax.transpose(jnp.min(dist_f, axis=1, keepdims=True), (1, 0)),
        jnp.min(dist_f, axis=0, keepdims=True),
        lax.transpose(jnp.min(dist_g, axis=1, keepdims=True), (1, 0)),
        jnp.min(dist_g, axis=0, keepdims=True),
    ], axis=0)
    hits = _topk_onehots(nnv4)

    fh, fl = _split16(dist_f)
    gh, gl = _split16(dist_g)
    zas = _split16(za)
    zbs = _split16(zb)
    for p, (dh, dl, ddim, zin, zcand) in enumerate((
            (fh, fl, 0, zas, zbs), (fh, fl, 1, zbs, zas),
            (gh, gl, 0, zas, zbs), (gh, gl, 1, zbs, zas))):
        s = jnp.concatenate([h[p:p + 1] for h in hits], axis=0)
        xin, xcand = _select_pairs(s, dh, dl, zin, zcand, ddim)
        fa_ref[p, i] = xin
        na_ref[p, i] = xcand


def _var_loss(xc, n):
    var = jnp.sum(xc * xc, axis=0, keepdims=True) / (n - 1.0)
    std = jnp.sqrt(var + EPS)
    return jnp.mean(jnp.maximum(1.0 - std, 0.0))


def _cov_loss_direct(xc, n, d):
    xh, xl = _split16(xc)
    m = _ddot(xh, xh, _TN) + _ddot(xh, xl, _TN) + _ddot(xl, xh, _TN)
    msq = m * m
    diag = lax.broadcasted_iota(jnp.int32, msq.shape, 0) == \
        lax.broadcasted_iota(jnp.int32, msq.shape, 1)
    off = jnp.sum(jnp.where(diag, 0.0, msq))
    return off / ((n - 1.0) ** 2 * d)


def _cov_loss_gram(xc, n, d):
    xh, xl = _split16(xc)
    g = _ddot(xh, xh, _NT) + _ddot(xh, xl, _NT) + _ddot(xl, xh, _NT)
    s = jnp.sum(xc * xc, axis=0, keepdims=True)
    off = jnp.sum(g * g) - jnp.sum(s * s)
    return off / ((n - 1.0) ** 2 * d)


def _vicreg_pair(xa, xb, n, d, gram):
    inv = jnp.sum((xa - xb) ** 2) / (n * d)
    xca = xa - jnp.mean(xa, axis=0, keepdims=True)
    xcb = xb - jnp.mean(xb, axis=0, keepdims=True)
    var = 0.5 * (_var_loss(xca, n) + _var_loss(xcb, n))
    covf = _cov_loss_gram if gram else _cov_loss_direct
    cov = covf(xca, n, d) + covf(xcb, n, d)
    return LAMBDA_PARAM * inv + MU_PARAM * var + NU_PARAM * cov


def _vicreg_kernel(fa_ref, na_ref, g0_ref, g1_ref, out_ref, acc_ref):
    p = pl.program_id(0)

    @pl.when(p == 0)
    def _():
        acc_ref[0] = ALPHA * _vicreg_pair(g0_ref[...], g1_ref[...], 64.0, float(D), True)

    n = float(64 * K)
    acc_ref[0] += (1.0 - ALPHA) * 0.5 * _vicreg_pair(
        fa_ref[0], na_ref[0], n, float(C), False)

    @pl.when(p == 3)
    def _():
        out_ref[0] = acc_ref[0]


@jax.jit
def _vicregl(z_global0, z_global1, z_local0, z_local1, grid0, grid1):
    B = z_local0.shape[0]
    za = z_local0.reshape(B, L, C)
    zb = z_local1.reshape(B, L, C)
    ga = grid0.reshape(B, L, 2)
    gb = grid1.reshape(B, L, 2)

    fa, na = pl.pallas_call(
        _match_kernel,
        grid=(B // IPS,),
        in_specs=[
            pl.BlockSpec((IPS, L, C), lambda b: (b, 0, 0)),
            pl.BlockSpec((IPS, L, C), lambda b: (b, 0, 0)),
            pl.BlockSpec((IPS, L, 2), lambda b: (b, 0, 0)),
            pl.BlockSpec((IPS, L, 2), lambda b: (b, 0, 0)),
        ],
        out_specs=[
            pl.BlockSpec((4, IPS, K, C), lambda b: (0, b, 0, 0)),
            pl.BlockSpec((4, IPS, K, C), lambda b: (0, b, 0, 0)),
        ],
        out_shape=[
            jax.ShapeDtypeStruct((4, B, K, C), jnp.float32),
            jax.ShapeDtypeStruct((4, B, K, C), jnp.float32),
        ],
    )(za, zb, ga, gb)

    fa = fa.reshape(4, B * K, C)
    na = na.reshape(4, B * K, C)

    out = pl.pallas_call(
        _vicreg_kernel,
        grid=(4,),
        in_specs=[
            pl.BlockSpec((1, B * K, C), lambda p: (p, 0, 0)),
            pl.BlockSpec((1, B * K, C), lambda p: (p, 0, 0)),
            pl.BlockSpec((B, D), lambda p: (0, 0)),
            pl.BlockSpec((B, D), lambda p: (0, 0)),
        ],
        out_specs=pl.BlockSpec(memory_space=pltpu.SMEM),
        out_shape=jax.ShapeDtypeStruct((1,), jnp.float32),
        scratch_shapes=[pltpu.SMEM((1,), jnp.float32)],
    )(fa, na, z_global0, z_global1)

    return out[0]


def kernel(z_global0, z_global1, z_local0, z_local1, grid0, grid1):
    return _vicregl(z_global0, z_global1, z_local0, z_local1, grid0, grid1)


# speedup vs baseline: 1.9322x; 1.2296x over previous
import functools

import jax
import jax.numpy as jnp
from jax import lax
from jax.experimental import pallas as pl
from jax.experimental.pallas import tpu as pltpu
from jax.experimental.pallas import tpu_sc as plsc

LAMBDA_PARAM = 25.0
MU_PARAM = 25.0
NU_PARAM = 1.0
ALPHA = 0.75
EPS = 1e-4
K = 20
L = 576
C = 768
D = 2048
BIG = 3.0e9

_NT = (((1,), (1,)), ((), ()))
_TN = (((0,), (0,)), ((), ()))


def _fiota(shape, dim):
    return lax.broadcasted_iota(jnp.int32, shape, dim).astype(jnp.float32)


def _dot(a, b, dims):
    return lax.dot_general(a, b, dims, precision=lax.Precision.HIGHEST)


def _ddot(a, b, dims):
    return lax.dot_general(a, b, dims, precision=lax.Precision.DEFAULT)


def _split16(x):
    hi = x.astype(jnp.bfloat16).astype(jnp.float32)
    return hi, x - hi


def _sdot(a, bh, bl, dims):
    return _ddot(a, bh, dims) + _ddot(a, bl, dims)


def _cdist_sq(za, zb):
    a2 = jnp.sum(za * za, axis=1, keepdims=True)
    ones = jnp.ones((1, za.shape[1]), jnp.float32)
    b2t = _dot(ones, zb * zb, _NT)
    d2 = a2 + b2t - 2.0 * lax.dot_general(za, zb, _NT,
                                          precision=lax.Precision.DEFAULT)
    return jnp.maximum(d2, 1e-12)


def _topk_onehots(nnv4):
    col = _fiota((4, L), 1)
    vals = nnv4
    hits, idxs = [], []
    for _ in range(K):
        m = jnp.min(vals, axis=1, keepdims=True)
        idx = jnp.min(jnp.where(vals == m, col, BIG), axis=1, keepdims=True)
        hit = col == idx
        hits.append(hit.astype(jnp.float32))
        idxs.append(idx)
        vals = jnp.where(hit, BIG, vals)
    return hits, jnp.concatenate(idxs, axis=1)


def _select_cand(s, disth, distl, ddim):
    dims = (((1,), (ddim,)), ((), ()))
    dsel = _ddot(s, disth, dims) + _ddot(s, distl, dims)
    nnv_sel = jnp.min(dsel, axis=1, keepdims=True)
    kcol = _fiota((K, L), 1)
    cand_f = jnp.min(jnp.where(dsel == nnv_sel, kcol, BIG), axis=1, keepdims=True)
    return lax.transpose(cand_f, (1, 0))


IPS = 2


def _match_kernel(za_ref, zb_ref, ga_ref, gb_ref, idx_ref):
    for i in range(IPS):
        _match_one(za_ref[i], zb_ref[i], ga_ref[i], gb_ref[i], idx_ref, i)


def _match_one(za, zb, ga, gb, idx_ref, i):
    dist_f = _cdist_sq(za, zb)
    dist_g = _cdist_sq(ga, gb)

    nnv4 = jnp.concatenate([
        lax.transpose(jnp.min(dist_f, axis=1, keepdims=True), (1, 0)),
        jnp.min(dist_f, axis=0, keepdims=True),
        lax.transpose(jnp.min(dist_g, axis=1, keepdims=True), (1, 0)),
        jnp.min(dist_g, axis=0, keepdims=True),
    ], axis=0)
    hits, in_idx4 = _topk_onehots(nnv4)

    img = pl.program_id(0) * IPS + i
    base = (img * L).astype(jnp.float32)

    fh, fl = _split16(dist_f)
    gh, gl = _split16(dist_g)
    for p, (dh, dl, ddim) in enumerate((
            (fh, fl, 0), (fh, fl, 1), (gh, gl, 0), (gh, gl, 1))):
        s = jnp.concatenate([h[p:p + 1] for h in hits], axis=0)
        cand = _select_cand(s, dh, dl, ddim)
        both = jnp.concatenate([in_idx4[p:p + 1], cand], axis=0) + base
        idx_ref[p, i] = both.astype(jnp.int32)


def _var_loss(xc, n):
    var = jnp.sum(xc * xc, axis=0, keepdims=True) / (n - 1.0)
    std = jnp.sqrt(var + EPS)
    return jnp.mean(jnp.maximum(1.0 - std, 0.0))


def _cov_loss_direct(xc, n, d):
    xh, xl = _split16(xc)
    m = _ddot(xh, xh, _TN) + _ddot(xh, xl, _TN) + _ddot(xl, xh, _TN)
    msq = m * m
    diag = lax.broadcasted_iota(jnp.int32, msq.shape, 0) == \
        lax.broadcasted_iota(jnp.int32, msq.shape, 1)
    off = jnp.sum(jnp.where(diag, 0.0, msq))
    return off / ((n - 1.0) ** 2 * d)


def _cov_loss_gram(xc, n, d):
    xh, xl = _split16(xc)
    g = _ddot(xh, xh, _NT) + _ddot(xh, xl, _NT) + _ddot(xl, xh, _NT)
    s = jnp.sum(xc * xc, axis=0, keepdims=True)
    off = jnp.sum(g * g) - jnp.sum(s * s)
    return off / ((n - 1.0) ** 2 * d)


def _vicreg_pair(xa, xb, n, d, gram):
    inv = jnp.sum((xa - xb) ** 2) / (n * d)
    xca = xa - jnp.mean(xa, axis=0, keepdims=True)
    xcb = xb - jnp.mean(xb, axis=0, keepdims=True)
    var = 0.5 * (_var_loss(xca, n) + _var_loss(xcb, n))
    covf = _cov_loss_gram if gram else _cov_loss_direct
    cov = covf(xca, n, d) + covf(xcb, n, d)
    return LAMBDA_PARAM * inv + MU_PARAM * var + NU_PARAM * cov


def _vicreg_kernel(fa_ref, na_ref, g0_ref, g1_ref, out_ref, acc_ref):
    p = pl.program_id(0)

    @pl.when(p == 0)
    def _():
        acc_ref[0] = ALPHA * _vicreg_pair(g0_ref[...], g1_ref[...], 64.0, float(D), True)

    n = float(64 * K)
    acc_ref[0] += (1.0 - ALPHA) * 0.5 * _vicreg_pair(
        fa_ref[0], na_ref[0], n, float(C), False)

    @pl.when(p == 3)
    def _():
        out_ref[0] = acc_ref[0]


_NC, _NS = 2, 16
_NW = _NC * _NS


def _sc_gather(z0, z1, gidx, B):
    n = B * K
    rpw = n // _NW

    @functools.partial(
        pl.kernel,
        mesh=plsc.VectorSubcoreMesh(core_axis_name="c", subcore_axis_name="s"),
        out_type=[
            jax.ShapeDtypeStruct((4, n, C), jnp.float32),
            jax.ShapeDtypeStruct((4, n, C), jnp.float32),
        ],
        scratch_types=[
            pltpu.VMEM((rpw,), jnp.int32),
            pltpu.VMEM((rpw, C), jnp.float32),
            pltpu.SemaphoreType.DMA,
        ],
    )
    def k(z0_hbm, z1_hbm, gidx_hbm, fa_hbm, na_hbm, idx_v, rows_v, sem):
        wid = lax.axis_index("s") * _NC + lax.axis_index("c")
        base = wid * rpw
        for p in range(4):
            for side in range(2):
                tab = z0_hbm if (side == 0) == (p % 2 == 0) else z1_hbm
                out = fa_hbm if side == 0 else na_hbm
                seg = (2 * p + side) * n
                pltpu.sync_copy(gidx_hbm.at[pl.ds(seg + base, rpw)], idx_v)
                pltpu.async_copy(tab.at[idx_v], rows_v, sem).wait()
                pltpu.sync_copy(rows_v, out.at[p, pl.ds(base, rpw)])

    return k(z0, z1, gidx)


@jax.jit
def _vicregl(z_global0, z_global1, z_local0, z_local1, grid0, grid1):
    B = z_local0.shape[0]
    za = z_local0.reshape(B, L, C)
    zb = z_local1.reshape(B, L, C)
    ga = grid0.reshape(B, L, 2)
    gb = grid1.reshape(B, L, 2)

    idx = pl.pallas_call(
        _match_kernel,
        grid=(B // IPS,),
        in_specs=[
            pl.BlockSpec((IPS, L, C), lambda b: (b, 0, 0)),
            pl.BlockSpec((IPS, L, C), lambda b: (b, 0, 0)),
            pl.BlockSpec((IPS, L, 2), lambda b: (b, 0, 0)),
            pl.BlockSpec((IPS, L, 2), lambda b: (b, 0, 0)),
        ],
        out_specs=pl.BlockSpec((4, IPS, 2, K), lambda b: (0, b, 0, 0)),
        out_shape=jax.ShapeDtypeStruct((4, B, 2, K), jnp.int32),
    )(za, zb, ga, gb)

    gidx = idx.transpose(0, 2, 1, 3).reshape(8 * B * K)
    fa, na = _sc_gather(z_local0.reshape(B * L, C), z_local1.reshape(B * L, C),
                        gidx, B)
    out = pl.pallas_call(
        _vicreg_kernel,
        grid=(4,),
        in_specs=[
            pl.BlockSpec((1, B * K, C), lambda p: (p, 0, 0)),
            pl.BlockSpec((1, B * K, C), lambda p: (p, 0, 0)),
            pl.BlockSpec((B, D), lambda p: (0, 0)),
            pl.BlockSpec((B, D), lambda p: (0, 0)),
        ],
        out_specs=pl.BlockSpec(memory_space=pltpu.SMEM),
        out_shape=jax.ShapeDtypeStruct((1,), jnp.float32),
        scratch_shapes=[pltpu.SMEM((1,), jnp.float32)],
    )(fa, na, z_global0, z_global1)

    return out[0]


def kernel(z_global0, z_global1, z_local0, z_local1, grid0, grid1):
    return _vicregl(z_global0, z_global1, z_local0, z_local1, grid0, grid1)


# speedup vs baseline: 2.3389x; 1.2105x over previous
import functools

import jax
import jax.numpy as jnp
from jax import lax
from jax.experimental import pallas as pl
from jax.experimental.pallas import tpu as pltpu
from jax.experimental.pallas import tpu_sc as plsc

LAMBDA_PARAM = 25.0
MU_PARAM = 25.0
NU_PARAM = 1.0
ALPHA = 0.75
EPS = 1e-4
K = 20
L = 576
C = 768
D = 2048
BIG = 3.0e9

_NT = (((1,), (1,)), ((), ()))
_TN = (((0,), (0,)), ((), ()))


def _fiota(shape, dim):
    return lax.broadcasted_iota(jnp.int32, shape, dim).astype(jnp.float32)


def _dot(a, b, dims):
    return lax.dot_general(a, b, dims, precision=lax.Precision.HIGHEST)


def _ddot(a, b, dims):
    return lax.dot_general(a, b, dims, precision=lax.Precision.DEFAULT)


def _split16(x):
    hi = x.astype(jnp.bfloat16).astype(jnp.float32)
    return hi, x - hi


def _sdot(a, bh, bl, dims):
    return _ddot(a, bh, dims) + _ddot(a, bl, dims)


def _cdist_sq(za, zb):
    a2 = jnp.sum(za * za, axis=1, keepdims=True)
    ones = jnp.ones((1, za.shape[1]), jnp.float32)
    b2t = _dot(ones, zb * zb, _NT)
    d2 = a2 + b2t - 2.0 * lax.dot_general(za, zb, _NT,
                                          precision=lax.Precision.DEFAULT)
    return jnp.maximum(d2, 1e-12)


def _topk_onehots(nnv):
    col = _fiota(nnv.shape, 1)
    vals = nnv
    hits, idxs = [], []
    for _ in range(K):
        m = jnp.min(vals, axis=1, keepdims=True)
        idx = jnp.min(jnp.where(vals == m, col, BIG), axis=1, keepdims=True)
        hit = col == idx
        hits.append(hit.astype(jnp.float32))
        idxs.append(idx)
        vals = jnp.where(hit, BIG, vals)
    return hits, jnp.concatenate(idxs, axis=1)


def _select_cand(s, disth, distl, ddim):
    dims = (((1,), (ddim,)), ((), ()))
    dsel = _ddot(s, disth, dims) + _ddot(s, distl, dims)
    nnv_sel = jnp.min(dsel, axis=1, keepdims=True)
    kcol = _fiota((K, L), 1)
    cand_f = jnp.min(jnp.where(dsel == nnv_sel, kcol, BIG), axis=1, keepdims=True)
    return lax.transpose(cand_f, (1, 0))


IPS = 2


def _match_kernel(za_ref, zb_ref, ga_ref, gb_ref, idx_ref):
    dists = [(_cdist_sq(za_ref[i], zb_ref[i]), _cdist_sq(ga_ref[i], gb_ref[i]))
             for i in range(IPS)]
    nnv = jnp.concatenate([
        jnp.concatenate([
            lax.transpose(jnp.min(df, axis=1, keepdims=True), (1, 0)),
            jnp.min(df, axis=0, keepdims=True),
            lax.transpose(jnp.min(dg, axis=1, keepdims=True), (1, 0)),
            jnp.min(dg, axis=0, keepdims=True),
        ], axis=0) for df, dg in dists], axis=0)
    hits, in_idx = _topk_onehots(nnv)

    for i, (dist_f, dist_g) in enumerate(dists):
        img = pl.program_id(0) * IPS + i
        base = (img * L).astype(jnp.float32)
        fh, fl = _split16(dist_f)
        gh, gl = _split16(dist_g)
        for p, (dh, dl, ddim) in enumerate((
                (fh, fl, 0), (fh, fl, 1), (gh, gl, 0), (gh, gl, 1))):
            q = 4 * i + p
            s = jnp.concatenate([h[q:q + 1] for h in hits], axis=0)
            cand = _select_cand(s, dh, dl, ddim)
            both = jnp.concatenate([in_idx[q:q + 1], cand], axis=0) + base
            idx_ref[p, i] = both.astype(jnp.int32)


def _var_loss(xc, n):
    var = jnp.sum(xc * xc, axis=0, keepdims=True) / (n - 1.0)
    std = jnp.sqrt(var + EPS)
    return jnp.mean(jnp.maximum(1.0 - std, 0.0))


def _cov_loss_direct(xc, n, d):
    xh, xl = _split16(xc)
    m = _ddot(xh, xh, _TN) + _ddot(xh, xl, _TN) + _ddot(xl, xh, _TN)
    msq = m * m
    diag = lax.broadcasted_iota(jnp.int32, msq.shape, 0) == \
        lax.broadcasted_iota(jnp.int32, msq.shape, 1)
    off = jnp.sum(jnp.where(diag, 0.0, msq))
    return off / ((n - 1.0) ** 2 * d)


def _cov_loss_gram(xc, n, d):
    xh, xl = _split16(xc)
    g = _ddot(xh, xh, _NT) + _ddot(xh, xl, _NT) + _ddot(xl, xh, _NT)
    s = jnp.sum(xc * xc, axis=0, keepdims=True)
    off = jnp.sum(g * g) - jnp.sum(s * s)
    return off / ((n - 1.0) ** 2 * d)


def _vicreg_pair(xa, xb, n, d, gram):
    inv = jnp.sum((xa - xb) ** 2) / (n * d)
    xca = xa - jnp.mean(xa, axis=0, keepdims=True)
    xcb = xb - jnp.mean(xb, axis=0, keepdims=True)
    var = 0.5 * (_var_loss(xca, n) + _var_loss(xcb, n))
    covf = _cov_loss_gram if gram else _cov_loss_direct
    cov = covf(xca, n, d) + covf(xcb, n, d)
    return LAMBDA_PARAM * inv + MU_PARAM * var + NU_PARAM * cov


def _vicreg_kernel(fa_ref, na_ref, g0_ref, g1_ref, out_ref, acc_ref):
    p = pl.program_id(0)

    @pl.when(p == 0)
    def _():
        acc_ref[0] = ALPHA * _vicreg_pair(g0_ref[...], g1_ref[...], 64.0, float(D), True)

    n = float(64 * K)
    acc_ref[0] += (1.0 - ALPHA) * 0.5 * _vicreg_pair(
        fa_ref[0], na_ref[0], n, float(C), False)

    @pl.when(p == 3)
    def _():
        out_ref[0] = acc_ref[0]


_NC, _NS = 2, 16
_NW = _NC * _NS


def _sc_gather(z0, z1, gidx, B):
    n = B * K
    rpw = n // _NW

    @functools.partial(
        pl.kernel,
        mesh=plsc.VectorSubcoreMesh(core_axis_name="c", subcore_axis_name="s", num_cores=_NC),
        out_type=[
            jax.ShapeDtypeStruct((4, n, C), jnp.float32),
            jax.ShapeDtypeStruct((4, n, C), jnp.float32),
        ],
        scratch_types=[
            pltpu.VMEM((rpw,), jnp.int32),
            pltpu.VMEM((rpw, C), jnp.float32),
            pltpu.SemaphoreType.DMA,
        ],
    )
    def k(z0_hbm, z1_hbm, gidx_hbm, fa_hbm, na_hbm, idx_v, rows_v, sem):
        wid = lax.axis_index("s") * _NC + lax.axis_index("c")
        base = wid * rpw
        for p in range(4):
            for side in range(2):
                tab = z0_hbm if (side == 0) == (p % 2 == 0) else z1_hbm
                out = fa_hbm if side == 0 else na_hbm
                seg = (2 * p + side) * n
                pltpu.sync_copy(gidx_hbm.at[pl.ds(seg + base, rpw)], idx_v)
                pltpu.async_copy(tab.at[idx_v], rows_v, sem).wait()
                pltpu.sync_copy(rows_v, out.at[p, pl.ds(base, rpw)])

    return k(z0, z1, gidx)


@jax.jit
def _vicregl(z_global0, z_global1, z_local0, z_local1, grid0, grid1):
    B = z_local0.shape[0]
    za = z_local0.reshape(B, L, C)
    zb = z_local1.reshape(B, L, C)
    ga = grid0.reshape(B, L, 2)
    gb = grid1.reshape(B, L, 2)

    idx = pl.pallas_call(
        _match_kernel,
        grid=(B // IPS,),
        in_specs=[
            pl.BlockSpec((IPS, L, C), lambda b: (b, 0, 0)),
            pl.BlockSpec((IPS, L, C), lambda b: (b, 0, 0)),
            pl.BlockSpec((IPS, L, 2), lambda b: (b, 0, 0)),
            pl.BlockSpec((IPS, L, 2), lambda b: (b, 0, 0)),
        ],
        out_specs=pl.BlockSpec((4, IPS, 2, K), lambda b: (0, b, 0, 0)),
        out_shape=jax.ShapeDtypeStruct((4, B, 2, K), jnp.int32),
    )(za, zb, ga, gb)

    gidx = idx.transpose(0, 2, 1, 3).reshape(8 * B * K)
    fa, na = _sc_gather(z_local0.reshape(B * L, C), z_local1.reshape(B * L, C),
                        gidx, B)
    out = pl.pallas_call(
        _vicreg_kernel,
        grid=(4,),
        in_specs=[
            pl.BlockSpec((1, B * K, C), lambda p: (p, 0, 0)),
            pl.BlockSpec((1, B * K, C), lambda p: (p, 0, 0)),
            pl.BlockSpec((B, D), lambda p: (0, 0)),
            pl.BlockSpec((B, D), lambda p: (0, 0)),
        ],
        out_specs=pl.BlockSpec(memory_space=pltpu.SMEM),
        out_shape=jax.ShapeDtypeStruct((1,), jnp.float32),
        scratch_shapes=[pltpu.SMEM((1,), jnp.float32)],
    )(fa, na, z_global0, z_global1)

    return out[0]


def kernel(z_global0, z_global1, z_local0, z_local1, grid0, grid1):
    return _vicregl(z_global0, z_global1, z_local0, z_local1, grid0, grid1)


# speedup vs baseline: 2.3494x; 1.0045x over previous
import functools

import jax
import jax.numpy as jnp
from jax import lax
from jax.experimental import pallas as pl
from jax.experimental.pallas import tpu as pltpu
from jax.experimental.pallas import tpu_sc as plsc

LAMBDA_PARAM = 25.0
MU_PARAM = 25.0
NU_PARAM = 1.0
ALPHA = 0.75
EPS = 1e-4
K = 20
L = 576
C = 768
D = 2048
BIG = 3.0e9

_NT = (((1,), (1,)), ((), ()))
_TN = (((0,), (0,)), ((), ()))


def _fiota(shape, dim):
    return lax.broadcasted_iota(jnp.int32, shape, dim).astype(jnp.float32)


def _dot(a, b, dims):
    return lax.dot_general(a, b, dims, precision=lax.Precision.HIGHEST)


def _ddot(a, b, dims):
    return lax.dot_general(a, b, dims, precision=lax.Precision.DEFAULT)


def _split16(x):
    hi = x.astype(jnp.bfloat16).astype(jnp.float32)
    return hi, x - hi


def _sdot(a, bh, bl, dims):
    return _ddot(a, bh, dims) + _ddot(a, bl, dims)


def _cdist_sq(za, zb):
    a2 = jnp.sum(za * za, axis=1, keepdims=True)
    ones = jnp.ones((1, za.shape[1]), jnp.float32)
    b2t = _dot(ones, zb * zb, _NT)
    d2 = a2 + b2t - 2.0 * lax.dot_general(za, zb, _NT,
                                          precision=lax.Precision.DEFAULT)
    return jnp.maximum(d2, 1e-12)


def _topk_onehots(nnv):
    col = _fiota(nnv.shape, 1)
    vals = nnv
    hits, idxs = [], []
    for _ in range(K):
        m = jnp.min(vals, axis=1, keepdims=True)
        idx = jnp.min(jnp.where(vals == m, col, BIG), axis=1, keepdims=True)
        hit = col == idx
        hits.append(hit.astype(jnp.float32))
        idxs.append(idx)
        vals = jnp.where(hit, BIG, vals)
    return hits, jnp.concatenate(idxs, axis=1)


def _select_cand(s, disth, distl, ddim):
    dims = (((1,), (ddim,)), ((), ()))
    dsel = _ddot(s, disth, dims) + _ddot(s, distl, dims)
    nnv_sel = jnp.min(dsel, axis=1, keepdims=True)
    kcol = _fiota((K, L), 1)
    cand_f = jnp.min(jnp.where(dsel == nnv_sel, kcol, BIG), axis=1, keepdims=True)
    return lax.transpose(cand_f, (1, 0))


IPS = 2


def _match_kernel(za_ref, zb_ref, ga_ref, gb_ref, idx_ref):
    dists = [(_cdist_sq(za_ref[i], zb_ref[i]), _cdist_sq(ga_ref[i], gb_ref[i]))
             for i in range(IPS)]
    nnv = jnp.concatenate([
        jnp.concatenate([
            lax.transpose(jnp.min(df, axis=1, keepdims=True), (1, 0)),
            jnp.min(df, axis=0, keepdims=True),
            lax.transpose(jnp.min(dg, axis=1, keepdims=True), (1, 0)),
            jnp.min(dg, axis=0, keepdims=True),
        ], axis=0) for df, dg in dists], axis=0)
    hits, in_idx = _topk_onehots(nnv)

    for i, (dist_f, dist_g) in enumerate(dists):
        img = pl.program_id(0) * IPS + i
        base = (img * L).astype(jnp.float32)
        fh, fl = _split16(dist_f)
        gh, gl = _split16(dist_g)
        for p, (dh, dl, ddim) in enumerate((
                (fh, fl, 0), (fh, fl, 1), (gh, gl, 0), (gh, gl, 1))):
            q = 4 * i + p
            s = jnp.concatenate([h[q:q + 1] for h in hits], axis=0)
            cand = _select_cand(s, dh, dl, ddim)
            both = jnp.concatenate([in_idx[q:q + 1], cand], axis=0) + base
            idx_ref[p, i] = both.astype(jnp.int32)


def _var_loss(xc, n):
    var = jnp.sum(xc * xc, axis=0, keepdims=True) / (n - 1.0)
    std = jnp.sqrt(var + EPS)
    return jnp.mean(jnp.maximum(1.0 - std, 0.0))


def _cov_loss_direct(xc, n, d):
    xh, xl = _split16(xc)
    m = _ddot(xh, xh, _TN) + _ddot(xh, xl, _TN) + _ddot(xl, xh, _TN)
    msq = m * m
    diag = lax.broadcasted_iota(jnp.int32, msq.shape, 0) == \
        lax.broadcasted_iota(jnp.int32, msq.shape, 1)
    off = jnp.sum(jnp.where(diag, 0.0, msq))
    return off / ((n - 1.0) ** 2 * d)


def _cov_loss_gram(xc, n, d):
    xh, xl = _split16(xc)
    g = _ddot(xh, xh, _NT) + _ddot(xh, xl, _NT) + _ddot(xl, xh, _NT)
    s = jnp.sum(xc * xc, axis=0, keepdims=True)
    off = jnp.sum(g * g) - jnp.sum(s * s)
    return off / ((n - 1.0) ** 2 * d)


def _vicreg_pair(xa, xb, n, d, gram):
    inv = jnp.sum((xa - xb) ** 2) / (n * d)
    xca = xa - jnp.mean(xa, axis=0, keepdims=True)
    xcb = xb - jnp.mean(xb, axis=0, keepdims=True)
    var = 0.5 * (_var_loss(xca, n) + _var_loss(xcb, n))
    covf = _cov_loss_gram if gram else _cov_loss_direct
    cov = covf(xca, n, d) + covf(xcb, n, d)
    return LAMBDA_PARAM * inv + MU_PARAM * var + NU_PARAM * cov


def _vicreg_kernel(fa_ref, na_ref, g0_ref, g1_ref, out_ref, acc_ref):
    p = pl.program_id(0)

    @pl.when(p == 0)
    def _():
        acc_ref[0] = ALPHA * _vicreg_pair(g0_ref[...], g1_ref[...], 64.0, float(D), True)

    n = float(64 * K)
    acc_ref[0] += (1.0 - ALPHA) * 0.5 * _vicreg_pair(
        fa_ref[0], na_ref[0], n, float(C), False)

    @pl.when(p == 3)
    def _():
        out_ref[0] = acc_ref[0]


_NC, _NS = 2, 16
_NW = _NC * _NS


def _sc_gather(z0, z1, gidx, B):
    n = B * K
    rpw = n // _NW

    @functools.partial(
        pl.kernel,
        mesh=plsc.VectorSubcoreMesh(core_axis_name="c", subcore_axis_name="s",
                                    num_cores=_NC),
        out_type=[
            jax.ShapeDtypeStruct((4, n, C), jnp.float32),
            jax.ShapeDtypeStruct((4, n, C), jnp.float32),
        ],
        scratch_types=[
            pltpu.VMEM((8 * rpw,), jnp.int32),
            pltpu.VMEM((2, rpw, C), jnp.float32),
            pltpu.SemaphoreType.DMA,
            pltpu.SemaphoreType.DMA,
            pltpu.SemaphoreType.DMA,
            pltpu.SemaphoreType.DMA,
        ],
    )
    def k(z0_hbm, z1_hbm, gidx_hbm, fa_hbm, na_hbm, idx_v, rows_v,
          g0, g1, w0, w1):
        wid = lax.axis_index("s") * _NC + lax.axis_index("c")
        base = wid * rpw
        pltpu.sync_copy(gidx_hbm.at[pl.ds(wid * 8 * rpw, 8 * rpw)], idx_v)
        gsem = (g0, g1)
        wsem = (w0, w1)

        def tab_out(seg):
            p, side = divmod(seg, 2)
            tab = z0_hbm if (side == 0) == (p % 2 == 0) else z1_hbm
            out = fa_hbm if side == 0 else na_hbm
            return tab, out, p

        def issue_gather(seg):
            tab, _, _ = tab_out(seg)
            b = seg % 2
            return pltpu.async_copy(tab.at[idx_v.at[pl.ds(seg * rpw, rpw)]],
                                    rows_v.at[b], gsem[b])

        gh = {0: issue_gather(0)}
        wh = {}
        for seg in range(8):
            b = seg % 2
            _, out, p = tab_out(seg)
            gh[seg].wait()
            if seg >= 2:
                wh[seg - 2].wait()
            if seg < 7:
                gh[seg + 1] = issue_gather(seg + 1)
            wh[seg] = pltpu.async_copy(rows_v.at[b],
                                       out.at[p, pl.ds(base, rpw)], wsem[b])
        wh[6].wait()
        wh[7].wait()

    return k(z0, z1, gidx)


@jax.jit
def _vicregl(z_global0, z_global1, z_local0, z_local1, grid0, grid1):
    B = z_local0.shape[0]
    za = z_local0.reshape(B, L, C)
    zb = z_local1.reshape(B, L, C)
    ga = grid0.reshape(B, L, 2)
    gb = grid1.reshape(B, L, 2)

    idx = pl.pallas_call(
        _match_kernel,
        grid=(B // IPS,),
        in_specs=[
            pl.BlockSpec((IPS, L, C), lambda b: (b, 0, 0)),
            pl.BlockSpec((IPS, L, C), lambda b: (b, 0, 0)),
            pl.BlockSpec((IPS, L, 2), lambda b: (b, 0, 0)),
            pl.BlockSpec((IPS, L, 2), lambda b: (b, 0, 0)),
        ],
        out_specs=pl.BlockSpec((4, IPS, 2, K), lambda b: (0, b, 0, 0)),
        out_shape=jax.ShapeDtypeStruct((4, B, 2, K), jnp.int32),
    )(za, zb, ga, gb)

    gidx = (idx.transpose(0, 2, 1, 3).reshape(8, _NW, (B * K) // _NW)
            .transpose(1, 0, 2).reshape(-1))
    fa, na = _sc_gather(z_local0.reshape(B * L, C), z_local1.reshape(B * L, C),
                        gidx, B)
    out = pl.pallas_call(
        _vicreg_kernel,
        grid=(4,),
        in_specs=[
            pl.BlockSpec((1, B * K, C), lambda p: (p, 0, 0)),
            pl.BlockSpec((1, B * K, C), lambda p: (p, 0, 0)),
            pl.BlockSpec((B, D), lambda p: (0, 0)),
            pl.BlockSpec((B, D), lambda p: (0, 0)),
        ],
        out_specs=pl.BlockSpec(memory_space=pltpu.SMEM),
        out_shape=jax.ShapeDtypeStruct((1,), jnp.float32),
        scratch_shapes=[pltpu.SMEM((1,), jnp.float32)],
    )(fa, na, z_global0, z_global1)

    return out[0]


def kernel(z_global0, z_global1, z_local0, z_local1, grid0, grid1):
    return _vicregl(z_global0, z_global1, z_local0, z_local1, grid0, grid1)


# speedup vs baseline: 2.8719x; 1.2224x over previous
import functools

import jax
import jax.numpy as jnp
from jax import lax
from jax.experimental import pallas as pl
from jax.experimental.pallas import tpu as pltpu
from jax.experimental.pallas import tpu_sc as plsc

LAMBDA_PARAM = 25.0
MU_PARAM = 25.0
NU_PARAM = 1.0
ALPHA = 0.75
EPS = 1e-4
K = 20
L = 576
C = 768
D = 2048
BIG = 3.0e9

_NT = (((1,), (1,)), ((), ()))
_TN = (((0,), (0,)), ((), ()))


def _fiota(shape, dim):
    return lax.broadcasted_iota(jnp.int32, shape, dim).astype(jnp.float32)


def _dot(a, b, dims):
    return lax.dot_general(a, b, dims, precision=lax.Precision.HIGHEST)


def _ddot(a, b, dims):
    return lax.dot_general(a, b, dims, precision=lax.Precision.DEFAULT)


def _split16(x):
    hi = x.astype(jnp.bfloat16).astype(jnp.float32)
    return hi, x - hi


def _sdot(a, bh, bl, dims):
    return _ddot(a, bh, dims) + _ddot(a, bl, dims)


def _cdist_sq(za, zb):
    a2 = jnp.sum(za * za, axis=1, keepdims=True)
    ones = jnp.ones((1, za.shape[1]), jnp.float32)
    b2t = _dot(ones, zb * zb, _NT)
    d2 = a2 + b2t - 2.0 * lax.dot_general(za, zb, _NT,
                                          precision=lax.Precision.DEFAULT)
    return jnp.maximum(d2, 1e-12)


def _topk_onehots(nnv):
    col = _fiota(nnv.shape, 1)
    vals = nnv
    hits, idxs = [], []
    for _ in range(K):
        m = jnp.min(vals, axis=1, keepdims=True)
        idx = jnp.min(jnp.where(vals == m, col, BIG), axis=1, keepdims=True)
        hit = col == idx
        hits.append(hit.astype(jnp.float32))
        idxs.append(idx)
        vals = jnp.where(hit, BIG, vals)
    return hits, jnp.concatenate(idxs, axis=1)


def _select_cand(s, disth, distl, ddim):
    dims = (((1,), (ddim,)), ((), ()))
    dsel = _ddot(s, disth, dims) + _ddot(s, distl, dims)
    nnv_sel = jnp.min(dsel, axis=1, keepdims=True)
    kcol = _fiota((K, L), 1)
    cand_f = jnp.min(jnp.where(dsel == nnv_sel, kcol, BIG), axis=1, keepdims=True)
    return lax.transpose(cand_f, (1, 0))


IPS = 4


def _match_kernel(za_ref, zb_ref, ga_ref, gb_ref, idx_ref):
    dists = [(_cdist_sq(za_ref[i], zb_ref[i]), _cdist_sq(ga_ref[i], gb_ref[i]))
             for i in range(IPS)]
    nnv = jnp.concatenate([
        jnp.concatenate([
            lax.transpose(jnp.min(df, axis=1, keepdims=True), (1, 0)),
            jnp.min(df, axis=0, keepdims=True),
            lax.transpose(jnp.min(dg, axis=1, keepdims=True), (1, 0)),
            jnp.min(dg, axis=0, keepdims=True),
        ], axis=0) for df, dg in dists], axis=0)
    hits, in_idx = _topk_onehots(nnv)

    for i, (dist_f, dist_g) in enumerate(dists):
        img = pl.program_id(0) * IPS + i
        base = (img * L).astype(jnp.float32)
        fh, fl = _split16(dist_f)
        gh, gl = _split16(dist_g)
        for p, (dh, dl, ddim) in enumerate((
                (fh, fl, 0), (fh, fl, 1), (gh, gl, 0), (gh, gl, 1))):
            q = 4 * i + p
            s = jnp.concatenate([h[q:q + 1] for h in hits], axis=0)
            cand = _select_cand(s, dh, dl, ddim)
            both = jnp.concatenate([in_idx[q:q + 1], cand], axis=0) + base
            idx_ref[p, i] = both.astype(jnp.int32)


def _var_loss(xc, n):
    var = jnp.sum(xc * xc, axis=0, keepdims=True) / (n - 1.0)
    std = jnp.sqrt(var + EPS)
    return jnp.mean(jnp.maximum(1.0 - std, 0.0))


def _cov_loss_direct(xc, n, d):
    xh, xl = _split16(xc)
    m = _ddot(xh, xh, _TN) + _ddot(xh, xl, _TN) + _ddot(xl, xh, _TN)
    msq = m * m
    diag = lax.broadcasted_iota(jnp.int32, msq.shape, 0) == \
        lax.broadcasted_iota(jnp.int32, msq.shape, 1)
    off = jnp.sum(jnp.where(diag, 0.0, msq))
    return off / ((n - 1.0) ** 2 * d)


def _cov_loss_gram(xc, n, d):
    xh, xl = _split16(xc)
    g = _ddot(xh, xh, _NT) + _ddot(xh, xl, _NT) + _ddot(xl, xh, _NT)
    s = jnp.sum(xc * xc, axis=0, keepdims=True)
    off = jnp.sum(g * g) - jnp.sum(s * s)
    return off / ((n - 1.0) ** 2 * d)


def _vicreg_pair(xa, xb, n, d, gram):
    inv = jnp.sum((xa - xb) ** 2) / (n * d)
    xca = xa - jnp.mean(xa, axis=0, keepdims=True)
    xcb = xb - jnp.mean(xb, axis=0, keepdims=True)
    var = 0.5 * (_var_loss(xca, n) + _var_loss(xcb, n))
    covf = _cov_loss_gram if gram else _cov_loss_direct
    cov = covf(xca, n, d) + covf(xcb, n, d)
    return LAMBDA_PARAM * inv + MU_PARAM * var + NU_PARAM * cov


def _vicreg_kernel(fa_ref, na_ref, g0_ref, g1_ref, out_ref, acc_ref):
    p = pl.program_id(0)

    @pl.when(p == 0)
    def _():
        acc_ref[0] = ALPHA * _vicreg_pair(g0_ref[...], g1_ref[...], 64.0, float(D), True)

    n = float(64 * K)
    acc_ref[0] += (1.0 - ALPHA) * 0.5 * _vicreg_pair(
        fa_ref[0], na_ref[0], n, float(C), False)

    @pl.when(p == 3)
    def _():
        out_ref[0] = acc_ref[0]


_NC, _NS = 2, 16
_NW = _NC * _NS


def _sc_gather(z0, z1, gidx, B):
    n = B * K
    rpw = n // _NW

    @functools.partial(
        pl.kernel,
        mesh=plsc.VectorSubcoreMesh(core_axis_name="c", subcore_axis_name="s",
                                    num_cores=_NC),
        out_type=[
            jax.ShapeDtypeStruct((4, n, C), jnp.float32),
            jax.ShapeDtypeStruct((4, n, C), jnp.float32),
        ],
        scratch_types=[
            pltpu.VMEM((8 * rpw,), jnp.int32),
            pltpu.VMEM((2, rpw, C), jnp.float32),
            pltpu.SemaphoreType.DMA,
            pltpu.SemaphoreType.DMA,
            pltpu.SemaphoreType.DMA,
            pltpu.SemaphoreType.DMA,
        ],
    )
    def k(z0_hbm, z1_hbm, gidx_hbm, fa_hbm, na_hbm, idx_v, rows_v,
          g0, g1, w0, w1):
        wid = lax.axis_index("s") * _NC + lax.axis_index("c")
        base = wid * rpw
        pltpu.sync_copy(gidx_hbm.at[pl.ds(wid * 8 * rpw, 8 * rpw)], idx_v)
        gsem = (g0, g1)
        wsem = (w0, w1)

        def tab_out(seg):
            p, side = divmod(seg, 2)
            tab = z0_hbm if (side == 0) == (p % 2 == 0) else z1_hbm
            out = fa_hbm if side == 0 else na_hbm
            return tab, out, p

        def issue_gather(seg):
            tab, _, _ = tab_out(seg)
            b = seg % 2
            return pltpu.async_copy(tab.at[idx_v.at[pl.ds(seg * rpw, rpw)]],
                                    rows_v.at[b], gsem[b])

        gh = {0: issue_gather(0)}
        wh = {}
        for seg in range(8):
            b = seg % 2
            _, out, p = tab_out(seg)
            gh[seg].wait()
            if seg >= 2:
                wh[seg - 2].wait()
            if seg < 7:
                gh[seg + 1] = issue_gather(seg + 1)
            wh[seg] = pltpu.async_copy(rows_v.at[b],
                                       out.at[p, pl.ds(base, rpw)], wsem[b])
        wh[6].wait()
        wh[7].wait()

    return k(z0, z1, gidx)


@jax.jit
def _vicregl(z_global0, z_global1, z_local0, z_local1, grid0, grid1):
    B = z_local0.shape[0]
    za = z_local0.reshape(B, L, C)
    zb = z_local1.reshape(B, L, C)
    ga = grid0.reshape(B, L, 2)
    gb = grid1.reshape(B, L, 2)

    idx = pl.pallas_call(
        _match_kernel,
        grid=(B // IPS,),
        in_specs=[
            pl.BlockSpec((IPS, L, C), lambda b: (b, 0, 0)),
            pl.BlockSpec((IPS, L, C), lambda b: (b, 0, 0)),
            pl.BlockSpec((IPS, L, 2), lambda b: (b, 0, 0)),
            pl.BlockSpec((IPS, L, 2), lambda b: (b, 0, 0)),
        ],
        out_specs=pl.BlockSpec((4, IPS, 2, K), lambda b: (0, b, 0, 0)),
        out_shape=jax.ShapeDtypeStruct((4, B, 2, K), jnp.int32),
    )(za, zb, ga, gb)

    gidx = (idx.transpose(0, 2, 1, 3).reshape(8, _NW, (B * K) // _NW)
            .transpose(1, 0, 2).reshape(-1))
    fa, na = _sc_gather(z_local0.reshape(B * L, C), z_local1.reshape(B * L, C),
                        gidx, B)
    out = pl.pallas_call(
        _vicreg_kernel,
        grid=(4,),
        in_specs=[
            pl.BlockSpec((1, B * K, C), lambda p: (p, 0, 0)),
            pl.BlockSpec((1, B * K, C), lambda p: (p, 0, 0)),
            pl.BlockSpec((B, D), lambda p: (0, 0)),
            pl.BlockSpec((B, D), lambda p: (0, 0)),
        ],
        out_specs=pl.BlockSpec(memory_space=pltpu.SMEM),
        out_shape=jax.ShapeDtypeStruct((1,), jnp.float32),
        scratch_shapes=[pltpu.SMEM((1,), jnp.float32)],
    )(fa, na, z_global0, z_global1)

    return out[0]


def kernel(z_global0, z_global1, z_local0, z_local1, grid0, grid1):
    return _vicregl(z_global0, z_global1, z_local0, z_local1, grid0, grid1)


# speedup vs baseline: 2.8763x; 1.0015x over previous
import functools

import jax
import jax.numpy as jnp
from jax import lax
from jax.experimental import pallas as pl
from jax.experimental.pallas import tpu as pltpu
from jax.experimental.pallas import tpu_sc as plsc

LAMBDA_PARAM = 25.0
MU_PARAM = 25.0
NU_PARAM = 1.0
ALPHA = 0.75
EPS = 1e-4
K = 20
L = 576
C = 768
D = 2048
BIG = 3.0e9

_NT = (((1,), (1,)), ((), ()))
_TN = (((0,), (0,)), ((), ()))


def _fiota(shape, dim):
    return lax.broadcasted_iota(jnp.int32, shape, dim).astype(jnp.float32)


def _dot(a, b, dims):
    return lax.dot_general(a, b, dims, precision=lax.Precision.HIGHEST)


def _ddot(a, b, dims):
    return lax.dot_general(a, b, dims, precision=lax.Precision.DEFAULT)


def _split16(x):
    hi = x.astype(jnp.bfloat16).astype(jnp.float32)
    return hi, x - hi


def _sdot(a, bh, bl, dims):
    return _ddot(a, bh, dims) + _ddot(a, bl, dims)


def _cdist_sq(za, zb):
    a2 = jnp.sum(za * za, axis=1, keepdims=True)
    ones = jnp.ones((1, za.shape[1]), jnp.float32)
    b2t = _dot(ones, zb * zb, _NT)
    d2 = a2 + b2t - 2.0 * lax.dot_general(za, zb, _NT,
                                          precision=lax.Precision.DEFAULT)
    return jnp.maximum(d2, 1e-12)


def _topk_onehots(nnv):
    col = _fiota(nnv.shape, 1)
    vals = nnv
    hits, idxs = [], []
    for _ in range(K):
        m = jnp.min(vals, axis=1, keepdims=True)
        idx = jnp.min(jnp.where(vals == m, col, BIG), axis=1, keepdims=True)
        hit = col == idx
        hits.append(hit.astype(jnp.float32))
        idxs.append(idx)
        vals = jnp.where(hit, BIG, vals)
    return hits, jnp.concatenate(idxs, axis=1)


def _select_cand(s, disth, distl, ddim):
    dims = (((1,), (ddim,)), ((), ()))
    dsel = _ddot(s, disth, dims) + _ddot(s, distl, dims)
    nnv_sel = jnp.min(dsel, axis=1, keepdims=True)
    kcol = _fiota((K, L), 1)
    cand_f = jnp.min(jnp.where(dsel == nnv_sel, kcol, BIG), axis=1, keepdims=True)
    return lax.transpose(cand_f, (1, 0))


IPS = 4


def _match_kernel(za_ref, zb_ref, ga_ref, gb_ref, idx_ref):
    dists = [(_cdist_sq(za_ref[i], zb_ref[i]), _cdist_sq(ga_ref[i], gb_ref[i]))
             for i in range(IPS)]
    nnv = jnp.concatenate([
        jnp.concatenate([
            lax.transpose(jnp.min(df, axis=1, keepdims=True), (1, 0)),
            jnp.min(df, axis=0, keepdims=True),
            lax.transpose(jnp.min(dg, axis=1, keepdims=True), (1, 0)),
            jnp.min(dg, axis=0, keepdims=True),
        ], axis=0) for df, dg in dists], axis=0)
    hits, in_idx = _topk_onehots(nnv)

    rows = [[None] * 8 for _ in range(IPS)]
    for i, (dist_f, dist_g) in enumerate(dists):
        img = pl.program_id(0) * IPS + i
        base = (img * L).astype(jnp.float32)
        fh, fl = _split16(dist_f)
        gh, gl = _split16(dist_g)
        for p, (dh, dl, ddim) in enumerate((
                (fh, fl, 0), (fh, fl, 1), (gh, gl, 0), (gh, gl, 1))):
            q = 4 * i + p
            s = jnp.concatenate([h[q:q + 1] for h in hits], axis=0)
            cand = _select_cand(s, dh, dl, ddim)
            rows[i][2 * p] = in_idx[q:q + 1] + base
            rows[i][2 * p + 1] = cand + base
    for w in range(IPS // 2):
        seg_rows = jnp.concatenate([
            jnp.concatenate([rows[2 * w][s], rows[2 * w + 1][s]], axis=1)
            for s in range(8)], axis=0)
        idx_ref[w] = seg_rows.astype(jnp.int32)


def _var_loss(xc, n):
    var = jnp.sum(xc * xc, axis=0, keepdims=True) / (n - 1.0)
    std = jnp.sqrt(var + EPS)
    return jnp.mean(jnp.maximum(1.0 - std, 0.0))


def _cov_loss_direct(xc, n, d):
    xh, xl = _split16(xc)
    m = _ddot(xh, xh, _TN) + _ddot(xh, xl, _TN) + _ddot(xl, xh, _TN)
    msq = m * m
    diag = lax.broadcasted_iota(jnp.int32, msq.shape, 0) == \
        lax.broadcasted_iota(jnp.int32, msq.shape, 1)
    off = jnp.sum(jnp.where(diag, 0.0, msq))
    return off / ((n - 1.0) ** 2 * d)


def _cov_loss_gram(xc, n, d):
    xh, xl = _split16(xc)
    g = _ddot(xh, xh, _NT) + _ddot(xh, xl, _NT) + _ddot(xl, xh, _NT)
    s = jnp.sum(xc * xc, axis=0, keepdims=True)
    off = jnp.sum(g * g) - jnp.sum(s * s)
    return off / ((n - 1.0) ** 2 * d)


def _vicreg_pair(xa, xb, n, d, gram):
    inv = jnp.sum((xa - xb) ** 2) / (n * d)
    xca = xa - jnp.mean(xa, axis=0, keepdims=True)
    xcb = xb - jnp.mean(xb, axis=0, keepdims=True)
    var = 0.5 * (_var_loss(xca, n) + _var_loss(xcb, n))
    covf = _cov_loss_gram if gram else _cov_loss_direct
    cov = covf(xca, n, d) + covf(xcb, n, d)
    return LAMBDA_PARAM * inv + MU_PARAM * var + NU_PARAM * cov


def _vicreg_kernel(fa_ref, na_ref, g0_ref, g1_ref, out_ref, acc_ref):
    p = pl.program_id(0)

    @pl.when(p == 0)
    def _():
        acc_ref[0] = ALPHA * _vicreg_pair(g0_ref[...], g1_ref[...], 64.0, float(D), True)

    n = float(64 * K)
    acc_ref[0] += (1.0 - ALPHA) * 0.5 * _vicreg_pair(
        fa_ref[0], na_ref[0], n, float(C), False)

    @pl.when(p == 3)
    def _():
        out_ref[0] = acc_ref[0]


_NC, _NS = 2, 16
_NW = _NC * _NS


def _sc_gather(z0, z1, gidx, B):
    n = B * K
    rpw = n // _NW

    @functools.partial(
        pl.kernel,
        mesh=plsc.VectorSubcoreMesh(core_axis_name="c", subcore_axis_name="s",
                                    num_cores=_NC),
        out_type=[
            jax.ShapeDtypeStruct((4, n, C), jnp.float32),
            jax.ShapeDtypeStruct((4, n, C), jnp.float32),
        ],
        scratch_types=[
            pltpu.VMEM((8 * rpw,), jnp.int32),
            pltpu.VMEM((2, rpw, C), jnp.float32),
            pltpu.SemaphoreType.DMA,
            pltpu.SemaphoreType.DMA,
            pltpu.SemaphoreType.DMA,
            pltpu.SemaphoreType.DMA,
        ],
    )
    def k(z0_hbm, z1_hbm, gidx_hbm, fa_hbm, na_hbm, idx_v, rows_v,
          g0, g1, w0, w1):
        wid = lax.axis_index("s") * _NC + lax.axis_index("c")
        base = wid * rpw
        pltpu.sync_copy(gidx_hbm.at[pl.ds(wid * 8 * rpw, 8 * rpw)], idx_v)
        gsem = (g0, g1)
        wsem = (w0, w1)

        def tab_out(seg):
            p, side = divmod(seg, 2)
            tab = z0_hbm if (side == 0) == (p % 2 == 0) else z1_hbm
            out = fa_hbm if side == 0 else na_hbm
            return tab, out, p

        def issue_gather(seg):
            tab, _, _ = tab_out(seg)
            b = seg % 2
            return pltpu.async_copy(tab.at[idx_v.at[pl.ds(seg * rpw, rpw)]],
                                    rows_v.at[b], gsem[b])

        gh = {0: issue_gather(0)}
        wh = {}
        for seg in range(8):
            b = seg % 2
            _, out, p = tab_out(seg)
            gh[seg].wait()
            if seg >= 2:
                wh[seg - 2].wait()
            if seg < 7:
                gh[seg + 1] = issue_gather(seg + 1)
            wh[seg] = pltpu.async_copy(rows_v.at[b],
                                       out.at[p, pl.ds(base, rpw)], wsem[b])
        wh[6].wait()
        wh[7].wait()

    return k(z0, z1, gidx)


@jax.jit
def _vicregl(z_global0, z_global1, z_local0, z_local1, grid0, grid1):
    B = z_local0.shape[0]
    za = z_local0.reshape(B, L, C)
    zb = z_local1.reshape(B, L, C)
    ga = grid0.reshape(B, L, 2)
    gb = grid1.reshape(B, L, 2)

    idx = pl.pallas_call(
        _match_kernel,
        grid=(B // IPS,),
        in_specs=[
            pl.BlockSpec((IPS, L, C), lambda b: (b, 0, 0)),
            pl.BlockSpec((IPS, L, C), lambda b: (b, 0, 0)),
            pl.BlockSpec((IPS, L, 2), lambda b: (b, 0, 0)),
            pl.BlockSpec((IPS, L, 2), lambda b: (b, 0, 0)),
        ],
        out_specs=pl.BlockSpec((IPS // 2, 8, 2 * K), lambda b: (b, 0, 0)),
        out_shape=jax.ShapeDtypeStruct((_NW, 8, 2 * K), jnp.int32),
    )(za, zb, ga, gb)

    gidx = idx.reshape(-1)
    fa, na = _sc_gather(z_local0.reshape(B * L, C), z_local1.reshape(B * L, C),
                        gidx, B)
    out = pl.pallas_call(
        _vicreg_kernel,
        grid=(4,),
        in_specs=[
            pl.BlockSpec((1, B * K, C), lambda p: (p, 0, 0)),
            pl.BlockSpec((1, B * K, C), lambda p: (p, 0, 0)),
            pl.BlockSpec((B, D), lambda p: (0, 0)),
            pl.BlockSpec((B, D), lambda p: (0, 0)),
        ],
        out_specs=pl.BlockSpec(memory_space=pltpu.SMEM),
        out_shape=jax.ShapeDtypeStruct((1,), jnp.float32),
        scratch_shapes=[pltpu.SMEM((1,), jnp.float32)],
    )(fa, na, z_global0, z_global1)

    return out[0]


def kernel(z_global0, z_global1, z_local0, z_local1, grid0, grid1):
    return _vicregl(z_global0, z_global1, z_local0, z_local1, grid0, grid1)
